# baseline (speedup 1.0000x reference)
import jax
import jax.numpy as jnp
from jax import lax
from jax.experimental import pallas as pl
from jax.experimental.pallas import tpu as pltpu
from jax.experimental.pallas import tpu_sc as plsc

NUM_CORES = 2
NUM_SUBCORES = 16
NUM_WORKERS = NUM_CORES * NUM_SUBCORES
LANES = 16

BATCH = 16384
K = 64
N_PER = BATCH // NUM_WORKERS
CHUNK = 128
N_CHUNKS = N_PER // CHUNK
GROUPS = CHUNK // LANES


def _body(xs_hbm, xr_hbm, xw_hbm, s_hbm, r_hbm, w_hbm, out_hbm,
          idx_v, s_buf, r_buf, w_buf, out_v, sem):
  wid = lax.axis_index("s") * NUM_CORES + lax.axis_index("c")
  base = wid * N_PER

  for t, x_hbm in enumerate((xs_hbm, xr_hbm, xw_hbm)):
    for j in range(N_CHUNKS):
      pltpu.sync_copy(x_hbm.at[pl.ds(base + j * CHUNK, CHUNK)],
                      idx_v.at[t * N_CHUNKS + j])

  lane = lax.iota(jnp.int32, LANES)

  for j in range(N_CHUNKS):
    cs = pltpu.async_copy(s_hbm.at[idx_v.at[0 * N_CHUNKS + j]], s_buf, sem)
    cr = pltpu.async_copy(r_hbm.at[idx_v.at[1 * N_CHUNKS + j]], r_buf, sem)
    cw = pltpu.async_copy(w_hbm.at[idx_v.at[2 * N_CHUNKS + j]], w_buf, sem)
    cs.wait()
    cr.wait()
    cw.wait()

    def group_body(g, _, j=j):
      acc = jnp.zeros((LANES,), jnp.float32)
      for e in range(LANES):
        row = g * LANES + e
        part = jnp.zeros((LANES,), jnp.float32)
        for q in range(K // LANES):
          sv = s_buf[row, pl.ds(q * LANES, LANES)]
          rv = r_buf[row, pl.ds(q * LANES, LANES)]
          wv = w_buf[row, pl.ds(q * LANES, LANES)]
          part = part + (sv + rv) * wv
        for step in (1, 2, 4, 8):
          perm = lane ^ step
          part = part + jnp.take(part, perm)
        acc = jnp.where(lane == e, part, acc)
      out_v[pl.ds(j * CHUNK + g * LANES, LANES)] = (
          1.0 / (1.0 + jnp.exp(-acc)))
      return 0

    lax.fori_loop(0, GROUPS, group_body, 0)

  pltpu.sync_copy(out_v, out_hbm.at[pl.ds(base, N_PER)])


@jax.jit
def kernel(X, s_table, r_table, w_table):
  xs, xr, xw = X[:, 0], X[:, 1], X[:, 2]
  mesh = plsc.VectorSubcoreMesh(core_axis_name="c", subcore_axis_name="s")
  run = pl.kernel(
      _body,
      out_type=jax.ShapeDtypeStruct((BATCH,), jnp.float32),
      mesh=mesh,
      scratch_types=[
          pltpu.VMEM((3 * N_CHUNKS, CHUNK), jnp.int32),
          pltpu.VMEM((CHUNK, K), jnp.float32),
          pltpu.VMEM((CHUNK, K), jnp.float32),
          pltpu.VMEM((CHUNK, K), jnp.float32),
          pltpu.VMEM((N_PER,), jnp.float32),
          pltpu.SemaphoreType.DMA,
      ],
      compiler_params=pltpu.CompilerParams(use_tc_tiling_on_sc=False),
  )
  return run(xs, xr, xw, s_table, r_table, w_table)
